# baseline (speedup 1.0000x reference)
import functools

import jax
import jax.numpy as jnp
from jax import lax
from jax.experimental import pallas as pl
from jax.experimental.pallas import tpu as pltpu
from jax.experimental.pallas import tpu_sc as plsc

_NC = 2
_NS = 16
_NW = _NC * _NS

_ROWS = 400


def _deg_sc(dsti, ones, zeros, n):
    nw, nch, cb = dsti.shape
    wd = ones.shape[1]
    rpt = n // _NS
    mesh = plsc.VectorSubcoreMesh(core_axis_name="c", subcore_axis_name="s")

    @functools.partial(
        pl.kernel, mesh=mesh,
        out_type=jax.ShapeDtypeStruct((_NC, n, wd), jnp.float32),
        scratch_types=[
            pltpu.VMEM((nch, cb), jnp.int32),
            pltpu.VMEM((cb, wd), jnp.float32),
            pltpu.VMEM_SHARED((n, wd), jnp.float32),
        ],
    )
    def k(dsti_hbm, ones_hbm, zeros_hbm, out_hbm, dstv, onev, acc):
        c = lax.axis_index("c")
        s = lax.axis_index("s")
        w = s * _NC + c
        r0 = s * rpt
        pltpu.sync_copy(zeros_hbm.at[pl.ds(r0, rpt)], acc.at[pl.ds(r0, rpt)])
        pltpu.sync_copy(dsti_hbm.at[w], dstv)
        pltpu.sync_copy(ones_hbm, onev)
        plsc.subcore_barrier()

        def body(j, carry):
            pltpu.sync_copy(onev, acc.at[dstv.at[j]], add=True)
            return carry

        lax.fori_loop(0, nch, body, 0)
        plsc.subcore_barrier()
        pltpu.sync_copy(acc.at[pl.ds(r0, rpt)], out_hbm.at[c].at[pl.ds(r0, rpt)])

    return k(dsti, ones, zeros)


def _prop_sc(hp, srci, dsti, zeros):
    nw, nch, cb = srci.shape
    npad = zeros.shape[0]
    d = hp.shape[1]
    rpt = npad // _NS
    assert nch % 3 == 2, "3-slot pipeline below assumes nch = 3k+2"
    mesh = plsc.VectorSubcoreMesh(core_axis_name="c", subcore_axis_name="s")

    @functools.partial(
        pl.kernel, mesh=mesh,
        out_type=jax.ShapeDtypeStruct((_NC, npad, d), jnp.float32),
        scratch_types=[
            [pltpu.VMEM((1, cb), jnp.int32)] * 3,
            [pltpu.VMEM((1, cb), jnp.int32)] * 3,
            [pltpu.VMEM((cb, d), jnp.float32)] * 3,
            [pltpu.SemaphoreType.DMA] * 3,
            [pltpu.SemaphoreType.DMA] * 3,
            pltpu.VMEM_SHARED((npad, d), jnp.float32),
        ],
    )
    def k(hp_hbm, srci_hbm, dsti_hbm, zeros_hbm, out_hbm,
          sidx, didx, rows, semi, semr, acc):
        c = lax.axis_index("c")
        s = lax.axis_index("s")
        w = s * _NC + c
        r0 = s * rpt
        pltpu.sync_copy(zeros_hbm.at[pl.ds(r0, rpt)], acc.at[pl.ds(r0, rpt)])
        plsc.subcore_barrier()

        def idx_load(j, x):
            pltpu.async_copy(srci_hbm.at[w].at[pl.ds(j, 1)], sidx[x], semi[x])
            pltpu.async_copy(dsti_hbm.at[w].at[pl.ds(j, 1)], didx[x], semi[x])

        def gather(j, x):
            pltpu.make_async_copy(srci_hbm.at[w].at[pl.ds(j, 1)], sidx[x], semi[x]).wait()
            pltpu.make_async_copy(dsti_hbm.at[w].at[pl.ds(j, 1)], didx[x], semi[x]).wait()
            pltpu.async_copy(hp_hbm.at[sidx[x].at[0]], rows[x], semr[x])

        def scatter(j, x):
            pltpu.make_async_copy(hp_hbm.at[sidx[x].at[0]], rows[x], semr[x]).wait()
            pltpu.sync_copy(rows[x], acc.at[didx[x].at[0]], add=True)

        idx_load(0, 0)
        idx_load(1, 1)
        idx_load(2, 2)
        gather(0, 0)
        gather(1, 1)

        def body(i, carry):
            j = 3 * i

            def step(o, x):
                gather(j + o + 2, (x + 2) % 3)
                scatter(j + o, x)
                nxt = j + o + 3

                @pl.when(nxt < nch)
                def _():
                    idx_load(nxt, x)

            step(0, 0)
            step(1, 1)
            step(2, 2)
            return carry

        lax.fori_loop(0, nch // 3, body, 0)
        scatter(nch - 2, (nch - 2) % 3)
        scatter(nch - 1, (nch - 1) % 3)
        plsc.subcore_barrier()
        pltpu.sync_copy(acc.at[pl.ds(r0, rpt)], out_hbm.at[c].at[pl.ds(r0, rpt)])

    return k(hp, srci, dsti, zeros)


def _softplus(x):
    return jnp.log(1.0 + jnp.exp(-jnp.abs(x))) + jnp.maximum(x, 0.0)


def _prep_tc(degp, x):
    n, d = x.shape

    def body(degp_ref, x_ref, dinv_ref, h_ref):
        deg = degp_ref[0, :, 0:1] + degp_ref[1, :, 0:1] + 1.0
        dv = lax.rsqrt(deg)
        dinv_ref[...] = dv
        h_ref[...] = x_ref[...] * dv

    return pl.pallas_call(
        body,
        grid=(n // _ROWS,),
        in_specs=[
            pl.BlockSpec((2, _ROWS, 128), lambda i: (0, i, 0)),
            pl.BlockSpec((_ROWS, d), lambda i: (i, 0)),
        ],
        out_specs=[
            pl.BlockSpec((_ROWS, 1), lambda i: (i, 0)),
            pl.BlockSpec((_ROWS, d), lambda i: (i, 0)),
        ],
        out_shape=[
            jax.ShapeDtypeStruct((n, 1), jnp.float32),
            jax.ShapeDtypeStruct((n, d), jnp.float32),
        ],
    )(degp, x)


def _layer_tc(p, hprev, dinv, W, b):
    n, d = hprev.shape
    dout = W.shape[1]

    def body(p_ref, h_ref, dinv_ref, w_ref, b_ref, o_ref):
        dv = dinv_ref[...]
        g = (p_ref[0] + p_ref[1] + h_ref[...]) * dv
        y = jnp.dot(g, w_ref[...], preferred_element_type=jnp.float32) + b_ref[...]
        o_ref[...] = _softplus(y) * dv

    return pl.pallas_call(
        body,
        grid=(n // _ROWS,),
        in_specs=[
            pl.BlockSpec((2, _ROWS, d), lambda i: (0, i, 0)),
            pl.BlockSpec((_ROWS, d), lambda i: (i, 0)),
            pl.BlockSpec((_ROWS, 1), lambda i: (i, 0)),
            pl.BlockSpec((d, dout), lambda i: (0, 0)),
            pl.BlockSpec((1, dout), lambda i: (0, 0)),
        ],
        out_specs=pl.BlockSpec((_ROWS, dout), lambda i: (i, 0)),
        out_shape=jax.ShapeDtypeStruct((n, dout), jnp.float32),
    )(p, hprev, dinv, W, b)


def _final_tc(p, hprev, dinv, Wmu, bmu, Wls, bls, eps):
    n, d = hprev.shape
    kk = Wmu.shape[1]

    def body(p_ref, h_ref, dinv_ref, wmu_ref, bmu_ref, wls_ref, bls_ref, eps_ref,
             z_ref, pout_ref, mu_ref, ls_ref, var_ref):
        dv = dinv_ref[...]
        g = (p_ref[0] + p_ref[1] + h_ref[...]) * dv
        mu = jnp.dot(g, wmu_ref[...], preferred_element_type=jnp.float32) + bmu_ref[...]
        ls = jnp.dot(g, wls_ref[...], preferred_element_type=jnp.float32) + bls_ref[...]
        var = jnp.exp(ls)
        z = mu + jnp.sqrt(var) * eps_ref[...]
        zmax = jnp.max(z, axis=1, keepdims=True)
        ez = jnp.exp(z - zmax)
        pout = ez / jnp.sum(ez, axis=1, keepdims=True)
        z_ref[...] = z
        pout_ref[...] = pout
        mu_ref[...] = mu
        ls_ref[...] = ls
        var_ref[...] = var

    outs = pl.pallas_call(
        body,
        grid=(n // _ROWS,),
        in_specs=[
            pl.BlockSpec((2, _ROWS, d), lambda i: (0, i, 0)),
            pl.BlockSpec((_ROWS, d), lambda i: (i, 0)),
            pl.BlockSpec((_ROWS, 1), lambda i: (i, 0)),
            pl.BlockSpec((d, kk), lambda i: (0, 0)),
            pl.BlockSpec((1, kk), lambda i: (0, 0)),
            pl.BlockSpec((d, kk), lambda i: (0, 0)),
            pl.BlockSpec((1, kk), lambda i: (0, 0)),
            pl.BlockSpec((_ROWS, kk), lambda i: (i, 0)),
        ],
        out_specs=[pl.BlockSpec((_ROWS, kk), lambda i: (i, 0))] * 5,
        out_shape=[jax.ShapeDtypeStruct((n, kk), jnp.float32)] * 5,
    )(p, hprev, dinv, Wmu, bmu, Wls, bls, eps)
    return tuple(outs)


def kernel(x, edge_index, W0, b0, W1, b1, W2, b2, Wmu, bmu, Wls, bls, eps):
    n, d = x.shape
    e = edge_index.shape[1]
    ew = e // _NW
    cb = 80
    nch = ew // cb

    npad = ((n + 8 * _NS - 1) // (8 * _NS)) * (8 * _NS)
    src = edge_index[0].reshape(_NW, nch, cb)
    dst = edge_index[1].reshape(_NW, nch, cb)
    zeros = jnp.zeros((npad, d), jnp.float32)
    ones = jnp.ones((cb, d), jnp.float32)

    degp = _deg_sc(dst, ones, zeros, npad)
    dinv, h0 = _prep_tc(degp, x)

    p1 = _prop_sc(h0, src, dst, zeros)
    h1 = _layer_tc(p1, h0, dinv, W0, b0.reshape(1, -1))
    p2 = _prop_sc(h1, src, dst, zeros)
    h2 = _layer_tc(p2, h1, dinv, W1, b1.reshape(1, -1))
    p3 = _prop_sc(h2, src, dst, zeros)
    h3 = _layer_tc(p3, h2, dinv, W2, b2.reshape(1, -1))
    p4 = _prop_sc(h3, src, dst, zeros)

    return _final_tc(p4, h3, dinv, Wmu, bmu.reshape(1, -1), Wls, bls.reshape(1, -1), eps)


# speedup vs baseline: 1.2641x; 1.2641x over previous
import functools

import jax
import jax.numpy as jnp
from jax import lax
from jax.experimental import pallas as pl
from jax.experimental.pallas import tpu as pltpu
from jax.experimental.pallas import tpu_sc as plsc

_NC = 2
_NS = 16
_NW = _NC * _NS

_ROWS = 2000


def _deg_sc(dsti, ones, zeros, n):
    nw, nch, cb = dsti.shape
    wd = ones.shape[1]
    rpt = n // _NS
    mesh = plsc.VectorSubcoreMesh(core_axis_name="c", subcore_axis_name="s")

    @functools.partial(
        pl.kernel, mesh=mesh,
        out_type=jax.ShapeDtypeStruct((_NC, n, wd), jnp.float32),
        scratch_types=[
            pltpu.VMEM((nch, cb), jnp.int32),
            pltpu.VMEM((cb, wd), jnp.float32),
            pltpu.VMEM_SHARED((n, wd), jnp.float32),
        ],
    )
    def k(dsti_hbm, ones_hbm, zeros_hbm, out_hbm, dstv, onev, acc):
        c = lax.axis_index("c")
        s = lax.axis_index("s")
        w = s * _NC + c
        r0 = s * rpt
        pltpu.sync_copy(zeros_hbm.at[pl.ds(r0, rpt)], acc.at[pl.ds(r0, rpt)])
        pltpu.sync_copy(dsti_hbm.at[w], dstv)
        pltpu.sync_copy(ones_hbm, onev)
        plsc.subcore_barrier()

        def body(j, carry):
            pltpu.sync_copy(onev, acc.at[dstv.at[j]], add=True)
            return carry

        lax.fori_loop(0, nch, body, 0)
        plsc.subcore_barrier()
        pltpu.sync_copy(acc.at[pl.ds(r0, rpt)], out_hbm.at[c].at[pl.ds(r0, rpt)])

    return k(dsti, ones, zeros)


def _prop_sc(hp, srci, dsti, zeros):
    nw, nch, cb = srci.shape
    npad = zeros.shape[0]
    d = hp.shape[1]
    rpt = npad // _NS
    assert nch % 2 == 0, "double-buffered loop below assumes even nch"
    mesh = plsc.VectorSubcoreMesh(core_axis_name="c", subcore_axis_name="s")

    @functools.partial(
        pl.kernel, mesh=mesh,
        out_type=jax.ShapeDtypeStruct((_NC, npad, d), jnp.float32),
        scratch_types=[
            pltpu.VMEM((nch, cb), jnp.int32),
            pltpu.VMEM((1, cb), jnp.int32),
            pltpu.VMEM((1, cb), jnp.int32),
            pltpu.VMEM((cb, d), jnp.float32),
            pltpu.VMEM((cb, d), jnp.float32),
            pltpu.VMEM_SHARED((npad, d), jnp.float32),
            pltpu.SemaphoreType.DMA,
            pltpu.SemaphoreType.DMA,
            pltpu.SemaphoreType.DMA,
            pltpu.SemaphoreType.DMA,
        ],
    )
    def k(hp_hbm, srci_hbm, dsti_hbm, zeros_hbm, out_hbm,
          srcv, didx_a, didx_b, rows_a, rows_b, acc,
          semr_a, semr_b, semi_a, semi_b):
        c = lax.axis_index("c")
        s = lax.axis_index("s")
        w = s * _NC + c
        r0 = s * rpt
        pltpu.sync_copy(zeros_hbm.at[pl.ds(r0, rpt)], acc.at[pl.ds(r0, rpt)])
        pltpu.sync_copy(srci_hbm.at[w], srcv)
        plsc.subcore_barrier()

        def gather(j, rows, didx, semr, semi):
            pltpu.async_copy(hp_hbm.at[srcv.at[j]], rows, semr)
            pltpu.async_copy(dsti_hbm.at[w].at[pl.ds(j, 1)], didx, semi)

        def drain_scatter(j, rows, didx, semr, semi):
            pltpu.make_async_copy(hp_hbm.at[srcv.at[j]], rows, semr).wait()
            pltpu.make_async_copy(dsti_hbm.at[w].at[pl.ds(j, 1)], didx, semi).wait()
            pltpu.sync_copy(rows, acc.at[didx.at[0]], add=True)

        gather(0, rows_a, didx_a, semr_a, semi_a)

        def body(i, carry):
            j = 2 * i + 1
            gather(j, rows_b, didx_b, semr_b, semi_b)
            drain_scatter(j - 1, rows_a, didx_a, semr_a, semi_a)
            gather(j + 1, rows_a, didx_a, semr_a, semi_a)
            drain_scatter(j, rows_b, didx_b, semr_b, semi_b)
            return carry

        lax.fori_loop(0, (nch - 2) // 2, body, 0)
        gather(nch - 1, rows_b, didx_b, semr_b, semi_b)
        drain_scatter(nch - 2, rows_a, didx_a, semr_a, semi_a)
        drain_scatter(nch - 1, rows_b, didx_b, semr_b, semi_b)
        plsc.subcore_barrier()
        pltpu.sync_copy(acc.at[pl.ds(r0, rpt)], out_hbm.at[c].at[pl.ds(r0, rpt)])

    return k(hp, srci, dsti, zeros)


def _softplus(x):
    return jnp.log(1.0 + jnp.exp(-jnp.abs(x))) + jnp.maximum(x, 0.0)


def _prep_tc(degp, x):
    n, d = x.shape

    def body(degp_ref, x_ref, dinv_ref, h_ref):
        deg = degp_ref[0, :, 0:1] + degp_ref[1, :, 0:1] + 1.0
        dv = lax.rsqrt(deg)
        dinv_ref[...] = dv
        h_ref[...] = x_ref[...] * dv

    return pl.pallas_call(
        body,
        grid=(n // _ROWS,),
        in_specs=[
            pl.BlockSpec((2, _ROWS, 128), lambda i: (0, i, 0)),
            pl.BlockSpec((_ROWS, d), lambda i: (i, 0)),
        ],
        out_specs=[
            pl.BlockSpec((_ROWS, 1), lambda i: (i, 0)),
            pl.BlockSpec((_ROWS, d), lambda i: (i, 0)),
        ],
        out_shape=[
            jax.ShapeDtypeStruct((n, 1), jnp.float32),
            jax.ShapeDtypeStruct((n, d), jnp.float32),
        ],
    )(degp, x)


def _layer_tc(p, hprev, dinv, W, b):
    n, d = hprev.shape
    dout = W.shape[1]

    def body(p_ref, h_ref, dinv_ref, w_ref, b_ref, o_ref):
        dv = dinv_ref[...]
        g = (p_ref[0] + p_ref[1] + h_ref[...]) * dv
        y = jnp.dot(g, w_ref[...], preferred_element_type=jnp.float32) + b_ref[...]
        o_ref[...] = _softplus(y) * dv

    return pl.pallas_call(
        body,
        grid=(n // _ROWS,),
        in_specs=[
            pl.BlockSpec((2, _ROWS, d), lambda i: (0, i, 0)),
            pl.BlockSpec((_ROWS, d), lambda i: (i, 0)),
            pl.BlockSpec((_ROWS, 1), lambda i: (i, 0)),
            pl.BlockSpec((d, dout), lambda i: (0, 0)),
            pl.BlockSpec((1, dout), lambda i: (0, 0)),
        ],
        out_specs=pl.BlockSpec((_ROWS, dout), lambda i: (i, 0)),
        out_shape=jax.ShapeDtypeStruct((n, dout), jnp.float32),
    )(p, hprev, dinv, W, b)


def _final_tc(p, hprev, dinv, Wmu, bmu, Wls, bls, eps):
    n, d = hprev.shape
    kk = Wmu.shape[1]

    def body(p_ref, h_ref, dinv_ref, wmu_ref, bmu_ref, wls_ref, bls_ref, eps_ref,
             z_ref, pout_ref, mu_ref, ls_ref, var_ref):
        dv = dinv_ref[...]
        g = (p_ref[0] + p_ref[1] + h_ref[...]) * dv
        mu = jnp.dot(g, wmu_ref[...], preferred_element_type=jnp.float32) + bmu_ref[...]
        ls = jnp.dot(g, wls_ref[...], preferred_element_type=jnp.float32) + bls_ref[...]
        var = jnp.exp(ls)
        z = mu + jnp.sqrt(var) * eps_ref[...]
        zmax = jnp.max(z, axis=1, keepdims=True)
        ez = jnp.exp(z - zmax)
        pout = ez / jnp.sum(ez, axis=1, keepdims=True)
        z_ref[...] = z
        pout_ref[...] = pout
        mu_ref[...] = mu
        ls_ref[...] = ls
        var_ref[...] = var

    outs = pl.pallas_call(
        body,
        grid=(n // _ROWS,),
        in_specs=[
            pl.BlockSpec((2, _ROWS, d), lambda i: (0, i, 0)),
            pl.BlockSpec((_ROWS, d), lambda i: (i, 0)),
            pl.BlockSpec((_ROWS, 1), lambda i: (i, 0)),
            pl.BlockSpec((d, kk), lambda i: (0, 0)),
            pl.BlockSpec((1, kk), lambda i: (0, 0)),
            pl.BlockSpec((d, kk), lambda i: (0, 0)),
            pl.BlockSpec((1, kk), lambda i: (0, 0)),
            pl.BlockSpec((_ROWS, kk), lambda i: (i, 0)),
        ],
        out_specs=[pl.BlockSpec((_ROWS, kk), lambda i: (i, 0))] * 5,
        out_shape=[jax.ShapeDtypeStruct((n, kk), jnp.float32)] * 5,
    )(p, hprev, dinv, Wmu, bmu, Wls, bls, eps)
    return tuple(outs)


def kernel(x, edge_index, W0, b0, W1, b1, W2, b2, Wmu, bmu, Wls, bls, eps):
    n, d = x.shape
    e = edge_index.shape[1]
    ew = e // _NW
    cb = 100
    nch = ew // cb

    npad = ((n + 8 * _NS - 1) // (8 * _NS)) * (8 * _NS)
    src = edge_index[0].reshape(_NW, nch, cb)
    dst = edge_index[1].reshape(_NW, nch, cb)
    zeros = jnp.zeros((npad, d), jnp.float32)
    ones = jnp.ones((cb, d), jnp.float32)

    degp = _deg_sc(dst, ones, zeros, npad)
    dinv, h0 = _prep_tc(degp, x)

    p1 = _prop_sc(h0, src, dst, zeros)
    h1 = _layer_tc(p1, h0, dinv, W0, b0.reshape(1, -1))
    p2 = _prop_sc(h1, src, dst, zeros)
    h2 = _layer_tc(p2, h1, dinv, W1, b1.reshape(1, -1))
    p3 = _prop_sc(h2, src, dst, zeros)
    h3 = _layer_tc(p3, h2, dinv, W2, b2.reshape(1, -1))
    p4 = _prop_sc(h3, src, dst, zeros)

    return _final_tc(p4, h3, dinv, Wmu, bmu.reshape(1, -1), Wls, bls.reshape(1, -1), eps)


# speedup vs baseline: 1.3589x; 1.0749x over previous
import functools

import jax
import jax.numpy as jnp
from jax import lax
from jax.experimental import pallas as pl
from jax.experimental.pallas import tpu as pltpu
from jax.experimental.pallas import tpu_sc as plsc

_NC = 2
_NS = 16
_NW = _NC * _NS

_ROWS = 2000


def _deg_sc(dsti, ones, zeros, n):
    nw, nch, cb = dsti.shape
    wd = ones.shape[1]
    rpt = n // _NS
    mesh = plsc.VectorSubcoreMesh(core_axis_name="c", subcore_axis_name="s")

    @functools.partial(
        pl.kernel, mesh=mesh,
        out_type=jax.ShapeDtypeStruct((_NC, n, wd), jnp.float32),
        scratch_types=[
            pltpu.VMEM((nch, cb), jnp.int32),
            pltpu.VMEM((cb, wd), jnp.float32),
            pltpu.VMEM_SHARED((n, wd), jnp.float32),
        ],
    )
    def k(dsti_hbm, ones_hbm, zeros_hbm, out_hbm, dstv, onev, acc):
        c = lax.axis_index("c")
        s = lax.axis_index("s")
        w = s * _NC + c
        r0 = s * rpt
        pltpu.sync_copy(zeros_hbm.at[pl.ds(r0, rpt)], acc.at[pl.ds(r0, rpt)])
        pltpu.sync_copy(dsti_hbm.at[w], dstv)
        pltpu.sync_copy(ones_hbm, onev)
        plsc.subcore_barrier()

        def body(j, carry):
            pltpu.sync_copy(onev, acc.at[dstv.at[j]], add=True)
            return carry

        lax.fori_loop(0, nch, body, 0)
        plsc.subcore_barrier()
        pltpu.sync_copy(acc.at[pl.ds(r0, rpt)], out_hbm.at[c].at[pl.ds(r0, rpt)])

    return k(dsti, ones, zeros)


def _prop_sc(hp, srci, dsti, zeros):
    nw, nch, cb = srci.shape
    npad = zeros.shape[0]
    d = hp.shape[1]
    rpt = npad // _NS
    assert nch % 3 == 2, "3-slot pipeline below assumes nch = 3k+2"
    mesh = plsc.VectorSubcoreMesh(core_axis_name="c", subcore_axis_name="s")

    @functools.partial(
        pl.kernel, mesh=mesh,
        out_type=jax.ShapeDtypeStruct((_NC, npad, d), jnp.float32),
        scratch_types=[
            pltpu.VMEM((nch, cb), jnp.int32),
            pltpu.VMEM((3, cb), jnp.int32),
            [pltpu.VMEM((cb, d), jnp.float32)] * 3,
            [pltpu.SemaphoreType.DMA] * 3,
            [pltpu.SemaphoreType.DMA] * 3,
            pltpu.VMEM_SHARED((npad, d), jnp.float32),
        ],
    )
    def k(hp_hbm, srci_hbm, dsti_hbm, zeros_hbm, out_hbm,
          srcv, didx, rows, semr, semi, acc):
        c = lax.axis_index("c")
        s = lax.axis_index("s")
        w = s * _NC + c
        r0 = s * rpt
        pltpu.sync_copy(zeros_hbm.at[pl.ds(r0, rpt)], acc.at[pl.ds(r0, rpt)])
        pltpu.sync_copy(srci_hbm.at[w], srcv)
        plsc.subcore_barrier()

        def didx_load(j, x):
            pltpu.async_copy(dsti_hbm.at[w].at[pl.ds(j, 1)], didx.at[pl.ds(x, 1)], semi[x])

        def gather(j, x):
            pltpu.async_copy(hp_hbm.at[srcv.at[j]], rows[x], semr[x])

        def pos(j, x, prefetch):
            pltpu.make_async_copy(hp_hbm.at[srcv.at[j]], rows[x], semr[x]).wait()
            pltpu.make_async_copy(
                dsti_hbm.at[w].at[pl.ds(j, 1)], didx.at[pl.ds(x, 1)], semi[x]).wait()
            gather(j + 2, (x + 2) % 3)
            pltpu.sync_copy(rows[x], acc.at[didx.at[x]], add=True)
            if prefetch:
                @pl.when(j + 3 < nch)
                def _():
                    didx_load(j + 3, x)

        didx_load(0, 0)
        didx_load(1, 1)
        didx_load(2, 2)
        gather(0, 0)
        gather(1, 1)

        def body(i, carry):
            j = 3 * i
            pos(j, 0, True)
            pos(j + 1, 1, True)
            pos(j + 2, 2, True)
            return carry

        lax.fori_loop(0, (nch - 2) // 3, body, 0)

        def tail(j, x):
            pltpu.make_async_copy(hp_hbm.at[srcv.at[j]], rows[x], semr[x]).wait()
            pltpu.make_async_copy(
                dsti_hbm.at[w].at[pl.ds(j, 1)], didx.at[pl.ds(x, 1)], semi[x]).wait()
            pltpu.sync_copy(rows[x], acc.at[didx.at[x]], add=True)

        tail(nch - 2, (nch - 2) % 3)
        tail(nch - 1, (nch - 1) % 3)
        plsc.subcore_barrier()
        pltpu.sync_copy(acc.at[pl.ds(r0, rpt)], out_hbm.at[c].at[pl.ds(r0, rpt)])

    return k(hp, srci, dsti, zeros)


def _softplus(x):
    return jnp.log(1.0 + jnp.exp(-jnp.abs(x))) + jnp.maximum(x, 0.0)


def _prep_tc(degp, x):
    n, d = x.shape

    def body(degp_ref, x_ref, dinv_ref, h_ref):
        deg = degp_ref[0, :, 0:1] + degp_ref[1, :, 0:1] + 1.0
        dv = lax.rsqrt(deg)
        dinv_ref[...] = dv
        h_ref[...] = x_ref[...] * dv

    return pl.pallas_call(
        body,
        grid=(n // _ROWS,),
        in_specs=[
            pl.BlockSpec((2, _ROWS, 128), lambda i: (0, i, 0)),
            pl.BlockSpec((_ROWS, d), lambda i: (i, 0)),
        ],
        out_specs=[
            pl.BlockSpec((_ROWS, 1), lambda i: (i, 0)),
            pl.BlockSpec((_ROWS, d), lambda i: (i, 0)),
        ],
        out_shape=[
            jax.ShapeDtypeStruct((n, 1), jnp.float32),
            jax.ShapeDtypeStruct((n, d), jnp.float32),
        ],
    )(degp, x)


def _layer_tc(p, hprev, dinv, W, b):
    n, d = hprev.shape
    dout = W.shape[1]

    def body(p_ref, h_ref, dinv_ref, w_ref, b_ref, o_ref):
        dv = dinv_ref[...]
        g = (p_ref[0] + p_ref[1] + h_ref[...]) * dv
        y = jnp.dot(g, w_ref[...], preferred_element_type=jnp.float32) + b_ref[...]
        o_ref[...] = _softplus(y) * dv

    return pl.pallas_call(
        body,
        grid=(n // _ROWS,),
        in_specs=[
            pl.BlockSpec((2, _ROWS, d), lambda i: (0, i, 0)),
            pl.BlockSpec((_ROWS, d), lambda i: (i, 0)),
            pl.BlockSpec((_ROWS, 1), lambda i: (i, 0)),
            pl.BlockSpec((d, dout), lambda i: (0, 0)),
            pl.BlockSpec((1, dout), lambda i: (0, 0)),
        ],
        out_specs=pl.BlockSpec((_ROWS, dout), lambda i: (i, 0)),
        out_shape=jax.ShapeDtypeStruct((n, dout), jnp.float32),
    )(p, hprev, dinv, W, b)


def _final_tc(p, hprev, dinv, Wmu, bmu, Wls, bls, eps):
    n, d = hprev.shape
    kk = Wmu.shape[1]

    def body(p_ref, h_ref, dinv_ref, wmu_ref, bmu_ref, wls_ref, bls_ref, eps_ref,
             z_ref, pout_ref, mu_ref, ls_ref, var_ref):
        dv = dinv_ref[...]
        g = (p_ref[0] + p_ref[1] + h_ref[...]) * dv
        mu = jnp.dot(g, wmu_ref[...], preferred_element_type=jnp.float32) + bmu_ref[...]
        ls = jnp.dot(g, wls_ref[...], preferred_element_type=jnp.float32) + bls_ref[...]
        var = jnp.exp(ls)
        z = mu + jnp.sqrt(var) * eps_ref[...]
        zmax = jnp.max(z, axis=1, keepdims=True)
        ez = jnp.exp(z - zmax)
        pout = ez / jnp.sum(ez, axis=1, keepdims=True)
        z_ref[...] = z
        pout_ref[...] = pout
        mu_ref[...] = mu
        ls_ref[...] = ls
        var_ref[...] = var

    outs = pl.pallas_call(
        body,
        grid=(n // _ROWS,),
        in_specs=[
            pl.BlockSpec((2, _ROWS, d), lambda i: (0, i, 0)),
            pl.BlockSpec((_ROWS, d), lambda i: (i, 0)),
            pl.BlockSpec((_ROWS, 1), lambda i: (i, 0)),
            pl.BlockSpec((d, kk), lambda i: (0, 0)),
            pl.BlockSpec((1, kk), lambda i: (0, 0)),
            pl.BlockSpec((d, kk), lambda i: (0, 0)),
            pl.BlockSpec((1, kk), lambda i: (0, 0)),
            pl.BlockSpec((_ROWS, kk), lambda i: (i, 0)),
        ],
        out_specs=[pl.BlockSpec((_ROWS, kk), lambda i: (i, 0))] * 5,
        out_shape=[jax.ShapeDtypeStruct((n, kk), jnp.float32)] * 5,
    )(p, hprev, dinv, Wmu, bmu, Wls, bls, eps)
    return tuple(outs)


def kernel(x, edge_index, W0, b0, W1, b1, W2, b2, Wmu, bmu, Wls, bls, eps):
    n, d = x.shape
    e = edge_index.shape[1]
    ew = e // _NW
    cb = 80
    nch = ew // cb

    npad = ((n + 8 * _NS - 1) // (8 * _NS)) * (8 * _NS)
    src = edge_index[0].reshape(_NW, nch, cb)
    dst = edge_index[1].reshape(_NW, nch, cb)
    zeros = jnp.zeros((npad, d), jnp.float32)
    ones = jnp.ones((cb, d), jnp.float32)

    degp = _deg_sc(dst, ones, zeros, npad)
    dinv, h0 = _prep_tc(degp, x)

    p1 = _prop_sc(h0, src, dst, zeros)
    h1 = _layer_tc(p1, h0, dinv, W0, b0.reshape(1, -1))
    p2 = _prop_sc(h1, src, dst, zeros)
    h2 = _layer_tc(p2, h1, dinv, W1, b1.reshape(1, -1))
    p3 = _prop_sc(h2, src, dst, zeros)
    h3 = _layer_tc(p3, h2, dinv, W2, b2.reshape(1, -1))
    p4 = _prop_sc(h3, src, dst, zeros)

    return _final_tc(p4, h3, dinv, Wmu, bmu.reshape(1, -1), Wls, bls.reshape(1, -1), eps)


# speedup vs baseline: 1.3589x; 1.0000x over previous
import functools

import jax
import jax.numpy as jnp
from jax import lax
from jax.experimental import pallas as pl
from jax.experimental.pallas import tpu as pltpu
from jax.experimental.pallas import tpu_sc as plsc

_NC = 2
_NS = 16
_NW = _NC * _NS

_ROWS = 2000


def _deg_sc(dsti, ones, zeros, n):
    nw, nch, cb = dsti.shape
    wd = ones.shape[1]
    rpt = n // _NS
    mesh = plsc.VectorSubcoreMesh(core_axis_name="c", subcore_axis_name="s")

    @functools.partial(
        pl.kernel, mesh=mesh,
        out_type=jax.ShapeDtypeStruct((_NC, n, wd), jnp.float32),
        scratch_types=[
            pltpu.VMEM((nch, cb), jnp.int32),
            pltpu.VMEM((cb, wd), jnp.float32),
            pltpu.VMEM_SHARED((n, wd), jnp.float32),
        ],
    )
    def k(dsti_hbm, ones_hbm, zeros_hbm, out_hbm, dstv, onev, acc):
        c = lax.axis_index("c")
        s = lax.axis_index("s")
        w = s * _NC + c
        r0 = s * rpt
        pltpu.sync_copy(zeros_hbm.at[pl.ds(r0, rpt)], acc.at[pl.ds(r0, rpt)])
        pltpu.sync_copy(dsti_hbm.at[w], dstv)
        pltpu.sync_copy(ones_hbm, onev)
        plsc.subcore_barrier()

        def body(j, carry):
            pltpu.sync_copy(onev, acc.at[dstv.at[j]], add=True)
            return carry

        lax.fori_loop(0, nch, body, 0)
        plsc.subcore_barrier()
        pltpu.sync_copy(acc.at[pl.ds(r0, rpt)], out_hbm.at[c].at[pl.ds(r0, rpt)])

    return k(dsti, ones, zeros)


def _prop_sc(hp, srci, dsti, zeros):
    nw, nch, cb = srci.shape
    npad = zeros.shape[0]
    d = hp.shape[1]
    rpt = npad // _NS
    assert nch % 3 == 2, "3-slot pipeline below assumes nch = 3k+2"
    mesh = plsc.VectorSubcoreMesh(core_axis_name="c", subcore_axis_name="s")

    @functools.partial(
        pl.kernel, mesh=mesh,
        out_type=jax.ShapeDtypeStruct((_NC, npad, d), jnp.float32),
        scratch_types=[
            pltpu.VMEM((nch, cb), jnp.int32),
            pltpu.VMEM((3, cb), jnp.int32),
            [pltpu.VMEM((cb, d), jnp.float32)] * 3,
            [pltpu.SemaphoreType.DMA] * 3,
            [pltpu.SemaphoreType.DMA] * 3,
            [pltpu.SemaphoreType.DMA] * 3,
            pltpu.VMEM_SHARED((npad, d), jnp.float32),
        ],
    )
    def k(hp_hbm, srci_hbm, dsti_hbm, zeros_hbm, out_hbm,
          srcv, didx, rows, semr, semi, semw, acc):
        c = lax.axis_index("c")
        s = lax.axis_index("s")
        w = s * _NC + c
        r0 = s * rpt
        pltpu.sync_copy(zeros_hbm.at[pl.ds(r0, rpt)], acc.at[pl.ds(r0, rpt)])
        pltpu.sync_copy(srci_hbm.at[w], srcv)
        plsc.subcore_barrier()

        def didx_load(j, x):
            pltpu.async_copy(dsti_hbm.at[w].at[pl.ds(j, 1)], didx.at[pl.ds(x, 1)], semi[x])

        def gather(j, x):
            pltpu.async_copy(hp_hbm.at[srcv.at[j]], rows[x], semr[x])

        def wait_gather(j, x):
            pltpu.make_async_copy(hp_hbm.at[srcv.at[j]], rows[x], semr[x]).wait()
            pltpu.make_async_copy(
                dsti_hbm.at[w].at[pl.ds(j, 1)], didx.at[pl.ds(x, 1)], semi[x]).wait()

        def scatter_start(j, x):
            pltpu.async_copy(rows[x], acc.at[didx.at[x]], semw[x], add=True)

        def scatter_drain(x):
            pltpu.make_async_copy(rows[x], acc.at[didx.at[x]], semw[x]).wait()

        def pos(j, x):
            g = (x + 2) % 3
            scatter_drain(g)
            didx_load(j + 2, g)
            wait_gather(j, x)
            gather(j + 2, g)
            scatter_start(j, x)

        didx_load(0, 0)
        didx_load(1, 1)
        didx_load(2, 2)
        gather(0, 0)
        gather(1, 1)
        wait_gather(0, 0)
        gather(2, 2)
        scatter_start(0, 0)

        def body(i, carry):
            j = 3 * i
            pos(j + 1, 1)
            pos(j + 2, 2)
            pos(j + 3, 0)
            return carry

        lax.fori_loop(0, (nch - 5) // 3, body, 0)
        pos(nch - 4, 1)
        pos(nch - 3, 2)

        def tail(j, x):
            wait_gather(j, x)
            scatter_start(j, x)

        scatter_drain(2)
        tail(nch - 2, 0)
        tail(nch - 1, 1)
        scatter_drain(0)
        scatter_drain(1)
        plsc.subcore_barrier()
        pltpu.sync_copy(acc.at[pl.ds(r0, rpt)], out_hbm.at[c].at[pl.ds(r0, rpt)])

    return k(hp, srci, dsti, zeros)


def _softplus(x):
    return jnp.log(1.0 + jnp.exp(-jnp.abs(x))) + jnp.maximum(x, 0.0)


def _prep_tc(degp, x):
    n, d = x.shape

    def body(degp_ref, x_ref, dinv_ref, h_ref):
        deg = degp_ref[0, :, 0:1] + degp_ref[1, :, 0:1] + 1.0
        dv = lax.rsqrt(deg)
        dinv_ref[...] = dv
        h_ref[...] = x_ref[...] * dv

    return pl.pallas_call(
        body,
        grid=(n // _ROWS,),
        in_specs=[
            pl.BlockSpec((2, _ROWS, 128), lambda i: (0, i, 0)),
            pl.BlockSpec((_ROWS, d), lambda i: (i, 0)),
        ],
        out_specs=[
            pl.BlockSpec((_ROWS, 1), lambda i: (i, 0)),
            pl.BlockSpec((_ROWS, d), lambda i: (i, 0)),
        ],
        out_shape=[
            jax.ShapeDtypeStruct((n, 1), jnp.float32),
            jax.ShapeDtypeStruct((n, d), jnp.float32),
        ],
    )(degp, x)


def _layer_tc(p, hprev, dinv, W, b):
    n, d = hprev.shape
    dout = W.shape[1]

    def body(p_ref, h_ref, dinv_ref, w_ref, b_ref, o_ref):
        dv = dinv_ref[...]
        g = (p_ref[0] + p_ref[1] + h_ref[...]) * dv
        y = jnp.dot(g, w_ref[...], preferred_element_type=jnp.float32) + b_ref[...]
        o_ref[...] = _softplus(y) * dv

    return pl.pallas_call(
        body,
        grid=(n // _ROWS,),
        in_specs=[
            pl.BlockSpec((2, _ROWS, d), lambda i: (0, i, 0)),
            pl.BlockSpec((_ROWS, d), lambda i: (i, 0)),
            pl.BlockSpec((_ROWS, 1), lambda i: (i, 0)),
            pl.BlockSpec((d, dout), lambda i: (0, 0)),
            pl.BlockSpec((1, dout), lambda i: (0, 0)),
        ],
        out_specs=pl.BlockSpec((_ROWS, dout), lambda i: (i, 0)),
        out_shape=jax.ShapeDtypeStruct((n, dout), jnp.float32),
    )(p, hprev, dinv, W, b)


def _final_tc(p, hprev, dinv, Wmu, bmu, Wls, bls, eps):
    n, d = hprev.shape
    kk = Wmu.shape[1]

    def body(p_ref, h_ref, dinv_ref, wmu_ref, bmu_ref, wls_ref, bls_ref, eps_ref,
             z_ref, pout_ref, mu_ref, ls_ref, var_ref):
        dv = dinv_ref[...]
        g = (p_ref[0] + p_ref[1] + h_ref[...]) * dv
        mu = jnp.dot(g, wmu_ref[...], preferred_element_type=jnp.float32) + bmu_ref[...]
        ls = jnp.dot(g, wls_ref[...], preferred_element_type=jnp.float32) + bls_ref[...]
        var = jnp.exp(ls)
        z = mu + jnp.sqrt(var) * eps_ref[...]
        zmax = jnp.max(z, axis=1, keepdims=True)
        ez = jnp.exp(z - zmax)
        pout = ez / jnp.sum(ez, axis=1, keepdims=True)
        z_ref[...] = z
        pout_ref[...] = pout
        mu_ref[...] = mu
        ls_ref[...] = ls
        var_ref[...] = var

    outs = pl.pallas_call(
        body,
        grid=(n // _ROWS,),
        in_specs=[
            pl.BlockSpec((2, _ROWS, d), lambda i: (0, i, 0)),
            pl.BlockSpec((_ROWS, d), lambda i: (i, 0)),
            pl.BlockSpec((_ROWS, 1), lambda i: (i, 0)),
            pl.BlockSpec((d, kk), lambda i: (0, 0)),
            pl.BlockSpec((1, kk), lambda i: (0, 0)),
            pl.BlockSpec((d, kk), lambda i: (0, 0)),
            pl.BlockSpec((1, kk), lambda i: (0, 0)),
            pl.BlockSpec((_ROWS, kk), lambda i: (i, 0)),
        ],
        out_specs=[pl.BlockSpec((_ROWS, kk), lambda i: (i, 0))] * 5,
        out_shape=[jax.ShapeDtypeStruct((n, kk), jnp.float32)] * 5,
    )(p, hprev, dinv, Wmu, bmu, Wls, bls, eps)
    return tuple(outs)


def kernel(x, edge_index, W0, b0, W1, b1, W2, b2, Wmu, bmu, Wls, bls, eps):
    n, d = x.shape
    e = edge_index.shape[1]
    ew = e // _NW
    cb = 80
    nch = ew // cb

    npad = ((n + 8 * _NS - 1) // (8 * _NS)) * (8 * _NS)
    src = edge_index[0].reshape(_NW, nch, cb)
    dst = edge_index[1].reshape(_NW, nch, cb)
    zeros = jnp.zeros((npad, d), jnp.float32)
    ones = jnp.ones((cb, d), jnp.float32)

    degp = _deg_sc(dst, ones, zeros, npad)
    dinv, h0 = _prep_tc(degp, x)

    p1 = _prop_sc(h0, src, dst, zeros)
    h1 = _layer_tc(p1, h0, dinv, W0, b0.reshape(1, -1))
    p2 = _prop_sc(h1, src, dst, zeros)
    h2 = _layer_tc(p2, h1, dinv, W1, b1.reshape(1, -1))
    p3 = _prop_sc(h2, src, dst, zeros)
    h3 = _layer_tc(p3, h2, dinv, W2, b2.reshape(1, -1))
    p4 = _prop_sc(h3, src, dst, zeros)

    return _final_tc(p4, h3, dinv, Wmu, bmu.reshape(1, -1), Wls, bls.reshape(1, -1), eps)
